# baseline (speedup 1.0000x reference)
import jax
import jax.numpy as jnp
from jax import lax
from jax.experimental import pallas as pl
from jax.experimental.pallas import tpu as pltpu
from jax.experimental.pallas import tpu_sc as plsc

B, S, NCB, D = 4, 2048, 8, 1024
N = B * S
NC, NS = 2, 16
NW = NC * NS
T = 16
NCH = N // T + 1
CPW = -(-NCH // NW)
NCHP = CPW * NW
NPOS = NCHP * T


def _embed_body(ids_hbm, dst_hbm, na_hbm, text_hbm, audio_hbm, out_hbm,
                ids_v, dst_v, na_v, g0, g1, g2, g3, g4, acc,
                s0, s1, s2, s3, s4, st, soa, sot):
    gb = (g0, g1, g2, g3, g4)
    gs = (s0, s1, s2, s3, s4)
    cid = lax.axis_index("c")
    sid = lax.axis_index("s")
    wid = sid * NC + cid

    pltpu.sync_copy(ids_hbm.at[wid], ids_v)
    pltpu.sync_copy(dst_hbm.at[wid], dst_v)
    pltpu.sync_copy(na_hbm, na_v)
    a = na_v[pl.ds(0, 16)][0]
    nca = (a + T - 1) // T
    nct = (N - a + T - 1) // T
    nctot = nca + nct
    dw = nctot - wid
    n_w = jnp.where(dw > 0, (dw + NW - 1) // NW, 0)

    dummy_rows = out_hbm.at[pl.ds(0, T)]

    def tree4(dest, b0, b1, b2, b3, accumulate):
        def row(t, _):
            for kk in range(64):
                off = kk * 16
                v = ((b0[t, pl.ds(off, 16)] + b1[t, pl.ds(off, 16)])
                     + (b2[t, pl.ds(off, 16)] + b3[t, pl.ds(off, 16)]))
                if accumulate:
                    plsc.addupdate(dest.at[t, pl.ds(off, 16)], v)
                else:
                    dest[t, pl.ds(off, 16)] = v
            return 0
        lax.fori_loop(0, T, row, 0, unroll=False)

    def chunk(k, carry):
        ci = wid + k * NW

        @pl.when(ci < nca)
        def _():
            pend = {}
            for j in range(5):
                pend[j] = pltpu.async_copy(
                    audio_hbm.at[ids_v.at[k, j]], gb[j], gs[j])
            for j in range(4):
                pend[j].wait()
            @pl.when(k > 0)
            def _():
                pltpu.make_async_copy(dummy_rows, acc, soa).wait()
            tree4(acc, g0, g1, g2, g3, False)
            for j in range(5, NCB):
                pend[j] = pltpu.async_copy(
                    audio_hbm.at[ids_v.at[k, j]], gb[j - 5], gs[j - 5])
            for j in range(4, NCB):
                pend[j].wait()
            tree4(acc, g4, g0, g1, g2, True)
            pltpu.async_copy(acc, out_hbm.at[dst_v.at[k]], soa)

        @pl.when(ci >= nca)
        def _():
            prev2_text = (k > 1) & (ci - 2 * NW >= nca)

            @pl.when(k % 2 == 0)
            def _():
                @pl.when(prev2_text)
                def _():
                    pltpu.make_async_copy(dummy_rows, g0, sot).wait()
                pltpu.async_copy(text_hbm.at[ids_v.at[k, 0]], g0, st).wait()
                pltpu.async_copy(g0, out_hbm.at[dst_v.at[k]], sot)

            @pl.when(k % 2 == 1)
            def _():
                @pl.when(prev2_text)
                def _():
                    pltpu.make_async_copy(dummy_rows, g1, sot).wait()
                pltpu.async_copy(text_hbm.at[ids_v.at[k, 0]], g1, st).wait()
                pltpu.async_copy(g1, out_hbm.at[dst_v.at[k]], sot)

        return carry

    lax.fori_loop(0, n_w, chunk, 0, unroll=False)

    naw = jnp.where(nca > wid, (nca - wid + NW - 1) // NW, 0)

    @pl.when(naw > 0)
    def _():
        pltpu.make_async_copy(dummy_rows, acc, soa).wait()

    ntw = n_w - naw

    @pl.when(ntw > 0)
    def _():
        pltpu.make_async_copy(dummy_rows, g0, sot).wait()

    @pl.when(ntw > 1)
    def _():
        pltpu.make_async_copy(dummy_rows, g1, sot).wait()


@jax.jit
def _sc_embed(ids, dst, na, text_table, audio_table):
    mesh = plsc.VectorSubcoreMesh(core_axis_name="c", subcore_axis_name="s")
    run = pl.kernel(
        _embed_body,
        out_type=jax.ShapeDtypeStruct((N, D), jnp.float32),
        mesh=mesh,
        scratch_types=[
            pltpu.VMEM((CPW, NCB, T), jnp.int32),
            pltpu.VMEM((CPW, T), jnp.int32),
            pltpu.VMEM((16,), jnp.int32),
            pltpu.VMEM((T, D), jnp.float32),
            pltpu.VMEM((T, D), jnp.float32),
            pltpu.VMEM((T, D), jnp.float32),
            pltpu.VMEM((T, D), jnp.float32),
            pltpu.VMEM((T, D), jnp.float32),
            pltpu.VMEM((T, D), jnp.float32),
            pltpu.SemaphoreType.DMA,
            pltpu.SemaphoreType.DMA,
            pltpu.SemaphoreType.DMA,
            pltpu.SemaphoreType.DMA,
            pltpu.SemaphoreType.DMA,
            pltpu.SemaphoreType.DMA,
            pltpu.SemaphoreType.DMA,
            pltpu.SemaphoreType.DMA,
        ],
    )
    return run(ids, dst, na, text_table, audio_table)


def kernel(input_ids, audio_mask, text_table, audio_table, offsets):
    ii32 = input_ids.astype(jnp.int32)
    m = audio_mask.reshape(N).astype(jnp.int32)
    shifted = (ii32 * audio_mask[:, None, :].astype(jnp.bool_).astype(jnp.int32)
               + offsets.reshape(1, -1, 1).astype(jnp.int32))
    shifted_tm = shifted.transpose(0, 2, 1).reshape(N, NCB)
    tid_raw = ii32[:, 0, :].reshape(N)
    tok = jnp.arange(N, dtype=jnp.int32)

    a_total = m.sum()
    nca = (a_total + T - 1) // T
    text_start = nca * T
    posA = jnp.cumsum(m) - m
    posT = jnp.cumsum(1 - m) - (1 - m)
    pos = jnp.where(m == 1, posA, text_start + posT)

    ids8 = jnp.where((m == 1)[:, None], shifted_tm,
                     jnp.concatenate(
                         [tid_raw[:, None],
                          jnp.zeros((N, NCB - 1), jnp.int32)], axis=1))
    payload = jnp.concatenate([ids8, tok[:, None]], axis=1)
    inv = jnp.zeros((NPOS,), jnp.int32).at[pos].set(tok)

    q = jnp.arange(NPOS, dtype=jnp.int32)
    is_real = ((q < a_total)
               | ((q >= text_start) & (q < text_start + (N - a_total))))
    invr = jnp.where(is_real.reshape(NCHP, T), inv.reshape(NCHP, T),
                     jnp.broadcast_to(inv.reshape(NCHP, T)[:, 0:1],
                                      (NCHP, T)))
    Pr = payload[invr]

    Pw = Pr.reshape(CPW, NW, T, NCB + 1).transpose(1, 0, 3, 2)
    ids = Pw[:, :, :NCB, :]
    dst = Pw[:, :, NCB, :]
    na = jnp.full((16,), a_total, jnp.int32)

    out = _sc_embed(ids, dst, na, text_table, audio_table)
    return out.reshape(B, S, D)


# speedup vs baseline: 1.3933x; 1.3933x over previous
import jax
import jax.numpy as jnp
from jax import lax
from jax.experimental import pallas as pl
from jax.experimental.pallas import tpu as pltpu
from jax.experimental.pallas import tpu_sc as plsc

B, S, NCB, D = 4, 2048, 8, 1024
N = B * S
NC, NS = 2, 16
NW = NC * NS
T = 16
NCH = N // T + 1
CPW = -(-NCH // NW)
NCHP = CPW * NW
NPOS = NCHP * T


def _embed_body(ids_hbm, dst_hbm, na_hbm, text_hbm, audio_hbm, out_hbm,
                ids_v, dst_v, na_v, g0, g1, g2, g3, g4, acc,
                s0, s1, s2, s3, s4, st, soa, sot):
    gb = (g0, g1, g2, g3, g4)
    gs = (s0, s1, s2, s3, s4)
    cid = lax.axis_index("c")
    sid = lax.axis_index("s")
    wid = sid * NC + cid

    pltpu.sync_copy(ids_hbm.at[wid], ids_v)
    pltpu.sync_copy(dst_hbm.at[wid], dst_v)
    pltpu.sync_copy(na_hbm, na_v)
    a = na_v[pl.ds(0, 16)][0]
    nca = (a + T - 1) // T
    nct = (N - a + T - 1) // T
    nctot = nca + nct
    dw = nctot - wid
    n_w = jnp.where(dw > 0, (dw + NW - 1) // NW, 0)

    dummy_rows = out_hbm.at[pl.ds(0, T)]

    def pair(dest, b0, b1, accumulate):
        def row(t, _):
            for kk in range(64):
                off = kk * 16
                v = b0[t, pl.ds(off, 16)] + b1[t, pl.ds(off, 16)]
                if accumulate:
                    plsc.addupdate(dest.at[t, pl.ds(off, 16)], v)
                else:
                    dest[t, pl.ds(off, 16)] = v
            return 0
        lax.fori_loop(0, T, row, 0, unroll=False)

    def chunk(k, carry):
        ci = wid + k * NW

        @pl.when(ci < nca)
        def _():
            pend = {}
            for j in range(5):
                pend[j] = pltpu.async_copy(
                    audio_hbm.at[ids_v.at[k, j]], gb[j], gs[j])
            pend[0].wait()
            pend[1].wait()
            @pl.when(k > 0)
            def _():
                pltpu.make_async_copy(dummy_rows, acc, soa).wait()
            pair(acc, g0, g1, False)
            pend[5] = pltpu.async_copy(
                audio_hbm.at[ids_v.at[k, 5]], g0, gs[0])
            pend[6] = pltpu.async_copy(
                audio_hbm.at[ids_v.at[k, 6]], g1, gs[1])
            pend[2].wait()
            pend[3].wait()
            pair(acc, g2, g3, True)
            pend[7] = pltpu.async_copy(
                audio_hbm.at[ids_v.at[k, 7]], g2, gs[2])
            pend[4].wait()
            pend[5].wait()
            pair(acc, g4, g0, True)
            pend[6].wait()
            pend[7].wait()
            pair(acc, g1, g2, True)
            pltpu.async_copy(acc, out_hbm.at[dst_v.at[k]], soa)

        @pl.when(ci >= nca)
        def _():
            prev2_text = (k > 1) & (ci - 2 * NW >= nca)

            @pl.when(k % 2 == 0)
            def _():
                @pl.when(prev2_text)
                def _():
                    pltpu.make_async_copy(dummy_rows, g3, sot).wait()
                pltpu.async_copy(text_hbm.at[ids_v.at[k, 0]], g3, st).wait()
                pltpu.async_copy(g3, out_hbm.at[dst_v.at[k]], sot)

            @pl.when(k % 2 == 1)
            def _():
                @pl.when(prev2_text)
                def _():
                    pltpu.make_async_copy(dummy_rows, g4, sot).wait()
                pltpu.async_copy(text_hbm.at[ids_v.at[k, 0]], g4, st).wait()
                pltpu.async_copy(g4, out_hbm.at[dst_v.at[k]], sot)

        return carry

    lax.fori_loop(0, n_w, chunk, 0, unroll=False)

    naw = jnp.where(nca > wid, (nca - wid + NW - 1) // NW, 0)

    @pl.when(naw > 0)
    def _():
        pltpu.make_async_copy(dummy_rows, acc, soa).wait()

    ntw = n_w - naw

    @pl.when(ntw > 0)
    def _():
        pltpu.make_async_copy(dummy_rows, g3, sot).wait()

    @pl.when(ntw > 1)
    def _():
        pltpu.make_async_copy(dummy_rows, g4, sot).wait()


@jax.jit
def _sc_embed(ids, dst, na, text_table, audio_table):
    mesh = plsc.VectorSubcoreMesh(core_axis_name="c", subcore_axis_name="s")
    run = pl.kernel(
        _embed_body,
        out_type=jax.ShapeDtypeStruct((N, D), jnp.float32),
        mesh=mesh,
        scratch_types=[
            pltpu.VMEM((CPW, NCB, T), jnp.int32),
            pltpu.VMEM((CPW, T), jnp.int32),
            pltpu.VMEM((16,), jnp.int32),
            pltpu.VMEM((T, D), jnp.float32),
            pltpu.VMEM((T, D), jnp.float32),
            pltpu.VMEM((T, D), jnp.float32),
            pltpu.VMEM((T, D), jnp.float32),
            pltpu.VMEM((T, D), jnp.float32),
            pltpu.VMEM((T, D), jnp.float32),
            pltpu.SemaphoreType.DMA,
            pltpu.SemaphoreType.DMA,
            pltpu.SemaphoreType.DMA,
            pltpu.SemaphoreType.DMA,
            pltpu.SemaphoreType.DMA,
            pltpu.SemaphoreType.DMA,
            pltpu.SemaphoreType.DMA,
            pltpu.SemaphoreType.DMA,
        ],
    )
    return run(ids, dst, na, text_table, audio_table)


def kernel(input_ids, audio_mask, text_table, audio_table, offsets):
    ii32 = input_ids.astype(jnp.int32)
    m = audio_mask.reshape(N).astype(jnp.int32)
    shifted = (ii32 * audio_mask[:, None, :].astype(jnp.bool_).astype(jnp.int32)
               + offsets.reshape(1, -1, 1).astype(jnp.int32))
    shifted_tm = shifted.transpose(0, 2, 1).reshape(N, NCB)
    tid_raw = ii32[:, 0, :].reshape(N)
    tok = jnp.arange(N, dtype=jnp.int32)

    a_total = m.sum()
    nca = (a_total + T - 1) // T
    text_start = nca * T
    posA = jnp.cumsum(m) - m
    posT = jnp.cumsum(1 - m) - (1 - m)
    pos = jnp.where(m == 1, posA, text_start + posT)

    ids8 = jnp.where((m == 1)[:, None], shifted_tm,
                     jnp.concatenate(
                         [tid_raw[:, None],
                          jnp.zeros((N, NCB - 1), jnp.int32)], axis=1))
    payload = jnp.concatenate([ids8, tok[:, None]], axis=1)
    P = jnp.zeros((NPOS, NCB + 1), jnp.int32).at[pos].set(payload)

    q = jnp.arange(NPOS, dtype=jnp.int32)
    is_real = ((q < a_total)
               | ((q >= text_start) & (q < text_start + (N - a_total))))
    Pr = P.reshape(NCHP, T, NCB + 1)
    Pr = jnp.where(is_real.reshape(NCHP, T)[:, :, None], Pr,
                   jnp.broadcast_to(Pr[:, 0:1, :], Pr.shape))

    Pw = Pr.reshape(CPW, NW, T, NCB + 1).transpose(1, 0, 3, 2)
    ids = Pw[:, :, :NCB, :]
    dst = Pw[:, :, NCB, :]
    na = jnp.full((16,), a_total, jnp.int32)

    out = _sc_embed(ids, dst, na, text_table, audio_table)
    return out.reshape(B, S, D)


# speedup vs baseline: 1.4059x; 1.0090x over previous
import jax
import jax.numpy as jnp
from jax import lax
from jax.experimental import pallas as pl
from jax.experimental.pallas import tpu as pltpu
from jax.experimental.pallas import tpu_sc as plsc

B, S, NCB, D = 4, 2048, 8, 1024
N = B * S
NC, NS = 2, 16
NW = NC * NS
T = 16
NCH = N // T + 1
CPW = -(-NCH // NW)
NCHP = CPW * NW
NPOS = NCHP * T


def _embed_body(ids_hbm, dst_hbm, na_hbm, text_hbm, audio_hbm, out_hbm,
                ids_v, dst_v, na_v, g0, g1, g2, g3, acc, acc2,
                s0, s1, s2, s3, s4, st, soa, sot):
    gb = (g0, g1, g2, g3)
    gs = (s0, s1, s2, s3)
    cid = lax.axis_index("c")
    sid = lax.axis_index("s")
    wid = sid * NC + cid

    pltpu.sync_copy(ids_hbm.at[wid], ids_v)
    pltpu.sync_copy(dst_hbm.at[wid], dst_v)
    pltpu.sync_copy(na_hbm, na_v)
    a = na_v[pl.ds(0, 16)][0]
    nca = (a + T - 1) // T
    nct = (N - a + T - 1) // T
    nctot = nca + nct
    dw = nctot - wid
    n_w = jnp.where(dw > 0, (dw + NW - 1) // NW, 0)

    dummy_rows = out_hbm.at[pl.ds(0, T)]

    def pair(dest, b0, b1, accumulate):
        def row(t, _):
            for kk in range(64):
                off = kk * 16
                v = b0[t, pl.ds(off, 16)] + b1[t, pl.ds(off, 16)]
                if accumulate:
                    plsc.addupdate(dest.at[t, pl.ds(off, 16)], v)
                else:
                    dest[t, pl.ds(off, 16)] = v
            return 0
        lax.fori_loop(0, T, row, 0, unroll=False)

    def chunk(k, carry):
        ci = wid + k * NW

        def audio_body(accb, drain_prev):
            pend = {}
            for j in range(4):
                pend[j] = pltpu.async_copy(
                    audio_hbm.at[ids_v.at[k, j]], gb[j], gs[j])
            pend[0].wait()
            pend[1].wait()
            @pl.when(drain_prev)
            def _():
                pltpu.make_async_copy(dummy_rows, accb, soa).wait()
            pair(accb, g0, g1, False)
            pend[4] = pltpu.async_copy(
                audio_hbm.at[ids_v.at[k, 4]], g0, gs[0])
            pend[5] = pltpu.async_copy(
                audio_hbm.at[ids_v.at[k, 5]], g1, gs[1])
            pend[2].wait()
            pend[3].wait()
            pair(accb, g2, g3, True)
            pend[6] = pltpu.async_copy(
                audio_hbm.at[ids_v.at[k, 6]], g2, gs[2])
            pend[7] = pltpu.async_copy(
                audio_hbm.at[ids_v.at[k, 7]], g3, gs[3])
            pend[4].wait()
            pend[5].wait()
            pair(accb, g0, g1, True)
            pend[6].wait()
            pend[7].wait()
            pair(accb, g2, g3, True)
            pltpu.async_copy(accb, out_hbm.at[dst_v.at[k]], soa)

        @pl.when((ci < nca) & (k % 2 == 0))
        def _():
            audio_body(acc, k > 1)

        @pl.when((ci < nca) & (k % 2 == 1))
        def _():
            audio_body(acc2, k > 1)

        @pl.when(ci >= nca)
        def _():
            prev2_text = (k > 1) & (ci - 2 * NW >= nca)

            @pl.when(k % 2 == 0)
            def _():
                @pl.when(prev2_text)
                def _():
                    pltpu.make_async_copy(dummy_rows, g0, sot).wait()
                pltpu.async_copy(text_hbm.at[ids_v.at[k, 0]], g0, st).wait()
                pltpu.async_copy(g0, out_hbm.at[dst_v.at[k]], sot)

            @pl.when(k % 2 == 1)
            def _():
                @pl.when(prev2_text)
                def _():
                    pltpu.make_async_copy(dummy_rows, g1, sot).wait()
                pltpu.async_copy(text_hbm.at[ids_v.at[k, 0]], g1, st).wait()
                pltpu.async_copy(g1, out_hbm.at[dst_v.at[k]], sot)

        return carry

    lax.fori_loop(0, n_w, chunk, 0, unroll=False)

    naw = jnp.where(nca > wid, (nca - wid + NW - 1) // NW, 0)

    @pl.when(naw > 0)
    def _():
        pltpu.make_async_copy(dummy_rows, acc, soa).wait()

    @pl.when(naw > 1)
    def _():
        pltpu.make_async_copy(dummy_rows, acc2, soa).wait()

    ntw = n_w - naw

    @pl.when(ntw > 0)
    def _():
        pltpu.make_async_copy(dummy_rows, g0, sot).wait()

    @pl.when(ntw > 1)
    def _():
        pltpu.make_async_copy(dummy_rows, g1, sot).wait()


@jax.jit
def _sc_embed(ids, dst, na, text_table, audio_table):
    mesh = plsc.VectorSubcoreMesh(core_axis_name="c", subcore_axis_name="s")
    run = pl.kernel(
        _embed_body,
        out_type=jax.ShapeDtypeStruct((N, D), jnp.float32),
        mesh=mesh,
        scratch_types=[
            pltpu.VMEM((CPW, NCB, T), jnp.int32),
            pltpu.VMEM((CPW, T), jnp.int32),
            pltpu.VMEM((16,), jnp.int32),
            pltpu.VMEM((T, D), jnp.float32),
            pltpu.VMEM((T, D), jnp.float32),
            pltpu.VMEM((T, D), jnp.float32),
            pltpu.VMEM((T, D), jnp.float32),
            pltpu.VMEM((T, D), jnp.float32),
            pltpu.VMEM((T, D), jnp.float32),
            pltpu.SemaphoreType.DMA,
            pltpu.SemaphoreType.DMA,
            pltpu.SemaphoreType.DMA,
            pltpu.SemaphoreType.DMA,
            pltpu.SemaphoreType.DMA,
            pltpu.SemaphoreType.DMA,
            pltpu.SemaphoreType.DMA,
            pltpu.SemaphoreType.DMA,
        ],
    )
    return run(ids, dst, na, text_table, audio_table)


def kernel(input_ids, audio_mask, text_table, audio_table, offsets):
    ii32 = input_ids.astype(jnp.int32)
    m = audio_mask.reshape(N).astype(jnp.int32)
    shifted = (ii32 * audio_mask[:, None, :].astype(jnp.bool_).astype(jnp.int32)
               + offsets.reshape(1, -1, 1).astype(jnp.int32))
    shifted_tm = shifted.transpose(0, 2, 1).reshape(N, NCB)
    tid_raw = ii32[:, 0, :].reshape(N)
    tok = jnp.arange(N, dtype=jnp.int32)

    a_total = m.sum()
    nca = (a_total + T - 1) // T
    text_start = nca * T
    posA = jnp.cumsum(m) - m
    posT = jnp.cumsum(1 - m) - (1 - m)
    pos = jnp.where(m == 1, posA, text_start + posT)

    ids8 = jnp.where((m == 1)[:, None], shifted_tm,
                     jnp.concatenate(
                         [tid_raw[:, None],
                          jnp.zeros((N, NCB - 1), jnp.int32)], axis=1))
    payload = jnp.concatenate([ids8, tok[:, None]], axis=1)
    P = jnp.zeros((NPOS, NCB + 1), jnp.int32).at[pos].set(payload)

    q = jnp.arange(NPOS, dtype=jnp.int32)
    is_real = ((q < a_total)
               | ((q >= text_start) & (q < text_start + (N - a_total))))
    Pr = P.reshape(NCHP, T, NCB + 1)
    Pr = jnp.where(is_real.reshape(NCHP, T)[:, :, None], Pr,
                   jnp.broadcast_to(Pr[:, 0:1, :], Pr.shape))

    Pw = Pr.reshape(CPW, NW, T, NCB + 1).transpose(1, 0, 3, 2)
    ids = Pw[:, :, :NCB, :]
    dst = Pw[:, :, NCB, :]
    na = jnp.full((16,), a_total, jnp.int32)

    out = _sc_embed(ids, dst, na, text_table, audio_table)
    return out.reshape(B, S, D)
